# Initial kernel scaffold; baseline (speedup 1.0000x reference)
import jax
import jax.numpy as jnp
from jax.experimental import pallas as pl


def kernel(x, edge_index, W1, b1, W2, b2):
    raise NotImplementedError("write your pallas kernel here")


# speedup vs baseline: 32.1611x; 32.1611x over previous
import functools

import jax
import jax.numpy as jnp
from jax import lax
from jax.experimental import pallas as pl
from jax.experimental.pallas import tpu as pltpu
from jax.experimental.pallas import tpu_sc as plsc

N = 10000
E = 320000
D_IN = 128
H = 16
C = 40
CP = 48

NW = 32
N_PAD = 10240
ROWS_PER_TILE = N_PAD // 16
E_PAD = 327680
CHUNKS = E_PAD // (NW * 128)
N_DUMMY_ROWS = 128

_MESH = plsc.VectorSubcoreMesh(core_axis_name="c", subcore_axis_name="s")
_SC_PARAMS = pltpu.CompilerParams(use_tc_tiling_on_sc=False)


@functools.partial(
    pl.kernel,
    mesh=_MESH,
    out_type=jax.ShapeDtypeStruct((2, N_PAD), jnp.float32),
    compiler_params=_SC_PARAMS,
    scratch_types=[
        pltpu.VMEM((CHUNKS, 128), jnp.int32),
        pltpu.VMEM((128,), jnp.float32),
        pltpu.VMEM((ROWS_PER_TILE,), jnp.float32),
        pltpu.VMEM_SHARED((N_PAD,), jnp.float32),
    ],
)
def _sc_degree(dst_hbm, deg_out, idx_v, ones_v, stage_v, acc_sh):
    c = lax.axis_index("c")
    s = lax.axis_index("s")
    wid = s * 2 + c

    def zstage(i, carry):
        stage_v[pl.ds(i * 16, 16)] = jnp.zeros((16,), jnp.float32)
        return carry

    lax.fori_loop(0, ROWS_PER_TILE // 16, zstage, 0)
    pltpu.sync_copy(stage_v, acc_sh.at[pl.ds(s * ROWS_PER_TILE, ROWS_PER_TILE)])
    for k in range(8):
        ones_v[pl.ds(k * 16, 16)] = jnp.ones((16,), jnp.float32)
    plsc.subcore_barrier()

    pltpu.sync_copy(dst_hbm.at[wid], idx_v)

    def body(j, carry):
        pltpu.sync_copy(ones_v, acc_sh.at[idx_v.at[j]], add=True)
        return carry

    lax.fori_loop(0, CHUNKS, body, 0)
    plsc.subcore_barrier()

    pltpu.sync_copy(acc_sh.at[pl.ds(s * ROWS_PER_TILE, ROWS_PER_TILE)], stage_v)
    pltpu.sync_copy(stage_v, deg_out.at[c, pl.ds(s * ROWS_PER_TILE, ROWS_PER_TILE)])


def _make_sc_prop(width):
    @functools.partial(
        pl.kernel,
        mesh=_MESH,
        out_type=jax.ShapeDtypeStruct((2, N_PAD, width), jnp.float32),
        compiler_params=_SC_PARAMS,
        scratch_types=[
            pltpu.VMEM((CHUNKS, 128), jnp.int32),
            pltpu.VMEM((CHUNKS, 128), jnp.int32),
            pltpu.VMEM((128, width), jnp.float32),
            pltpu.VMEM_SHARED((N_PAD, width), jnp.float32),
            pltpu.SemaphoreType.DMA,
        ],
    )
    def _sc_prop(z_hbm, src_hbm, dst_hbm, out_hbm, si_v, di_v, gbuf, acc_sh, sem):
        c = lax.axis_index("c")
        s = lax.axis_index("s")
        wid = s * 2 + c

        def zbuf(i, carry):
            for k in range(width // 16):
                gbuf[i, pl.ds(k * 16, 16)] = jnp.zeros((16,), jnp.float32)
            return carry

        lax.fori_loop(0, 128, zbuf, 0)

        def zacc(i, carry):
            pltpu.sync_copy(
                gbuf, acc_sh.at[pl.ds(s * ROWS_PER_TILE + i * 128, 128)])
            return carry

        lax.fori_loop(0, ROWS_PER_TILE // 128, zacc, 0)
        pltpu.sync_copy(src_hbm.at[wid], si_v)
        pltpu.sync_copy(dst_hbm.at[wid], di_v)
        plsc.subcore_barrier()

        def body(j, carry):
            pltpu.async_copy(z_hbm.at[si_v.at[j]], gbuf, sem).wait()
            pltpu.sync_copy(gbuf, acc_sh.at[di_v.at[j]], add=True)
            return carry

        lax.fori_loop(0, CHUNKS, body, 0)
        plsc.subcore_barrier()

        def wback(i, carry):
            base = s * ROWS_PER_TILE + i * 128
            pltpu.sync_copy(acc_sh.at[pl.ds(base, 128)], gbuf)
            pltpu.sync_copy(gbuf, out_hbm.at[c, pl.ds(base, 128)])
            return carry

        lax.fori_loop(0, ROWS_PER_TILE // 128, wback, 0)

    return _sc_prop


_sc_prop16 = _make_sc_prop(H)
_sc_prop48 = _make_sc_prop(CP)


def _dinv_from(degT_ref):
    d = degT_ref[:, 0:1] + degT_ref[:, 1:2] + 1.0
    rows = lax.broadcasted_iota(jnp.int32, (N_PAD, 1), 0)
    return jnp.where(rows < N, lax.rsqrt(d), 0.0)


def _tc_layer1_body(x_ref, w1_ref, degT_ref, z1_ref):
    h = jnp.dot(x_ref[...], w1_ref[...], preferred_element_type=jnp.float32)
    z1_ref[...] = _dinv_from(degT_ref) * h


def _tc_layer2_body(s1_ref, z1_ref, degT_ref, b1_ref, w2_ref, z2_ref):
    dinv = _dinv_from(degT_ref)
    t = dinv * (s1_ref[0] + s1_ref[1] + z1_ref[...]) + b1_ref[...]
    out1 = jnp.maximum(t, 0.0)
    z2_ref[...] = dinv * jnp.dot(out1, w2_ref[...],
                                 preferred_element_type=jnp.float32)


def _tc_softmax_body(s2_ref, z2_ref, degT_ref, b2_ref, out_ref):
    dinv = _dinv_from(degT_ref)
    u = dinv * (s2_ref[0] + s2_ref[1] + z2_ref[...]) + b2_ref[...]
    cols = lax.broadcasted_iota(jnp.int32, (1, CP), 1)
    cmask = cols < C
    u = jnp.where(cmask, u, -jnp.inf)
    m = jnp.max(u, axis=1, keepdims=True)
    e = jnp.where(cmask, jnp.exp(u - m), 0.0)
    out_ref[...] = e / jnp.sum(e, axis=1, keepdims=True)


def kernel(x, edge_index, W1, b1, W2, b2):
    src = edge_index[0]
    dst = edge_index[1]
    npad_e = E_PAD - E
    fill = N + (jnp.arange(npad_e, dtype=jnp.int32) % N_DUMMY_ROWS)
    src3 = jnp.concatenate([src, fill]).reshape(NW, CHUNKS, 128)
    dst3 = jnp.concatenate([dst, fill]).reshape(NW, CHUNKS, 128)
    x_pad = jnp.zeros((N_PAD, D_IN), jnp.float32).at[:N].set(x)
    w2p = jnp.zeros((H, CP), jnp.float32).at[:, :C].set(W2)
    b1r = b1.reshape(1, H)
    b2r = jnp.zeros((1, CP), jnp.float32).at[0, :C].set(b2)

    deg = _sc_degree(dst3)
    degT = deg.T

    z1 = pl.pallas_call(
        _tc_layer1_body,
        out_shape=jax.ShapeDtypeStruct((N_PAD, H), jnp.float32),
    )(x_pad, W1, degT)

    s1 = _sc_prop16(z1, src3, dst3)

    z2 = pl.pallas_call(
        _tc_layer2_body,
        out_shape=jax.ShapeDtypeStruct((N_PAD, CP), jnp.float32),
    )(s1, z1, degT, b1r, w2p)

    s2 = _sc_prop48(z2, src3, dst3)

    p = pl.pallas_call(
        _tc_softmax_body,
        out_shape=jax.ShapeDtypeStruct((N_PAD, CP), jnp.float32),
    )(s2, z2, degT, b2r)

    return p[:N, :C]


# speedup vs baseline: 53.6267x; 1.6674x over previous
import functools

import jax
import jax.numpy as jnp
from jax import lax
from jax.experimental import pallas as pl
from jax.experimental.pallas import tpu as pltpu
from jax.experimental.pallas import tpu_sc as plsc

N = 10000
E = 320000
D_IN = 128
H = 16
C = 40
CP = 48

NW = 32
N_PAD = 10240
ROWS_PER_TILE = N_PAD // 16
E_PAD = 327680
CHUNKS = E_PAD // (NW * 128)
N_DUMMY_ROWS = 128

_MESH = plsc.VectorSubcoreMesh(core_axis_name="c", subcore_axis_name="s")
_SC_PARAMS = pltpu.CompilerParams(use_tc_tiling_on_sc=False)


@functools.partial(
    pl.kernel,
    mesh=_MESH,
    out_type=jax.ShapeDtypeStruct((2, N_PAD), jnp.float32),
    compiler_params=_SC_PARAMS,
    scratch_types=[
        pltpu.VMEM((CHUNKS, 128), jnp.int32),
        pltpu.VMEM((128,), jnp.float32),
        pltpu.VMEM((ROWS_PER_TILE,), jnp.float32),
        pltpu.VMEM_SHARED((N_PAD,), jnp.float32),
    ],
)
def _sc_degree(dst_hbm, deg_out, idx_v, ones_v, stage_v, acc_sh):
    c = lax.axis_index("c")
    s = lax.axis_index("s")
    wid = s * 2 + c

    def zstage(i, carry):
        stage_v[pl.ds(i * 16, 16)] = jnp.zeros((16,), jnp.float32)
        return carry

    lax.fori_loop(0, ROWS_PER_TILE // 16, zstage, 0)
    pltpu.sync_copy(stage_v, acc_sh.at[pl.ds(s * ROWS_PER_TILE, ROWS_PER_TILE)])
    for k in range(8):
        ones_v[pl.ds(k * 16, 16)] = jnp.ones((16,), jnp.float32)
    plsc.subcore_barrier()

    pltpu.sync_copy(dst_hbm.at[wid], idx_v)

    def body(j, carry):
        pltpu.sync_copy(ones_v, acc_sh.at[idx_v.at[j]], add=True)
        return carry

    lax.fori_loop(0, CHUNKS, body, 0)
    plsc.subcore_barrier()

    pltpu.sync_copy(acc_sh.at[pl.ds(s * ROWS_PER_TILE, ROWS_PER_TILE)], stage_v)
    pltpu.sync_copy(stage_v, deg_out.at[c, pl.ds(s * ROWS_PER_TILE, ROWS_PER_TILE)])


NBUF = 8
GA = 4


def _make_sc_prop(width):
    @functools.partial(
        pl.kernel,
        mesh=_MESH,
        out_type=jax.ShapeDtypeStruct((2, N_PAD, width), jnp.float32),
        compiler_params=_SC_PARAMS,
        scratch_types=[
            pltpu.VMEM((CHUNKS, 128), jnp.int32),
            pltpu.VMEM((CHUNKS, 128), jnp.int32),
            pltpu.VMEM((NBUF, 128, width), jnp.float32),
            pltpu.VMEM((128, width), jnp.float32),
            pltpu.VMEM_SHARED((N_PAD, width), jnp.float32),
            pltpu.SemaphoreType.DMA((NBUF,)),
            pltpu.SemaphoreType.DMA((NBUF,)),
        ],
    )
    def _sc_prop(z_hbm, src_hbm, dst_hbm, out_hbm, si_v, di_v, gbufs, zbuf,
                 acc_sh, semg, sems):
        c = lax.axis_index("c")
        s = lax.axis_index("s")
        wid = s * 2 + c

        ld_si = pltpu.async_copy(src_hbm.at[wid], si_v, semg.at[0])
        ld_di = pltpu.async_copy(dst_hbm.at[wid], di_v, semg.at[1])

        def zb(i, carry):
            for k in range(width // 16):
                zbuf[i, pl.ds(k * 16, 16)] = jnp.zeros((16,), jnp.float32)
            return carry

        lax.fori_loop(0, 128, zb, 0)

        def zacc(i, carry):
            pltpu.sync_copy(
                zbuf, acc_sh.at[pl.ds(s * ROWS_PER_TILE + i * 128, 128)])
            return carry

        lax.fori_loop(0, ROWS_PER_TILE // 128, zacc, 0)
        ld_si.wait()
        ld_di.wait()
        plsc.subcore_barrier()

        for b in range(GA):
            pltpu.async_copy(z_hbm.at[si_v.at[b]], gbufs.at[b], semg.at[b])

        def group(g, carry):
            for b in range(NBUF):
                j = g * NBUF + b
                pltpu.make_async_copy(
                    z_hbm.at[si_v.at[j]], gbufs.at[b], semg.at[b]).wait()
                pltpu.async_copy(
                    gbufs.at[b], acc_sh.at[di_v.at[j]], sems.at[b], add=True)
                bc = (b + GA) % NBUF

                def refill():
                    pltpu.make_async_copy(
                        gbufs.at[bc], acc_sh.at[di_v.at[j]],
                        sems.at[bc]).wait()
                    pltpu.async_copy(
                        z_hbm.at[si_v.at[j + GA]], gbufs.at[bc], semg.at[bc])

                if b < GA:
                    @pl.when(g > 0)
                    def _():
                        refill()

                    @pl.when(g == 0)
                    def _():
                        pltpu.async_copy(z_hbm.at[si_v.at[j + GA]],
                                         gbufs.at[bc], semg.at[bc])
                else:
                    @pl.when(g < CHUNKS // NBUF - 1)
                    def _():
                        refill()
            return carry

        lax.fori_loop(0, CHUNKS // NBUF, group, 0)
        for b in range(NBUF):
            pltpu.make_async_copy(
                gbufs.at[b], acc_sh.at[di_v.at[CHUNKS - 1]], sems.at[b]).wait()
        plsc.subcore_barrier()

        nwb = ROWS_PER_TILE // 128
        for i in range(nwb):
            base = s * ROWS_PER_TILE + i * 128
            pltpu.async_copy(acc_sh.at[pl.ds(base, 128)], gbufs.at[i],
                             semg.at[i])
        for i in range(nwb):
            base = s * ROWS_PER_TILE + i * 128
            pltpu.make_async_copy(acc_sh.at[pl.ds(base, 128)], gbufs.at[i],
                                  semg.at[i]).wait()
            pltpu.async_copy(gbufs.at[i], out_hbm.at[c, pl.ds(base, 128)],
                             sems.at[i])
        for i in range(nwb):
            base = s * ROWS_PER_TILE + i * 128
            pltpu.make_async_copy(gbufs.at[i], out_hbm.at[c, pl.ds(base, 128)],
                                  sems.at[i]).wait()

    return _sc_prop


_sc_prop16 = _make_sc_prop(H)
_sc_prop48 = _make_sc_prop(CP)


def _dinv_from(degT_ref):
    d = degT_ref[:, 0:1] + degT_ref[:, 1:2] + 1.0
    rows = lax.broadcasted_iota(jnp.int32, (N_PAD, 1), 0)
    return jnp.where(rows < N, lax.rsqrt(d), 0.0)


def _tc_layer1_body(x_ref, w1_ref, degT_ref, z1_ref):
    h = jnp.dot(x_ref[...], w1_ref[...], preferred_element_type=jnp.float32)
    dinv = _dinv_from(degT_ref)
    z1_ref[pl.ds(0, N), :] = dinv[:N, :] * h
    z1_ref[pl.ds(N, N_PAD - N), :] = jnp.zeros((N_PAD - N, H), jnp.float32)


def _tc_layer2_body(s1_ref, z1_ref, degT_ref, b1_ref, w2_ref, z2_ref):
    dinv = _dinv_from(degT_ref)
    t = dinv * (s1_ref[0] + s1_ref[1] + z1_ref[...]) + b1_ref[...]
    out1 = jnp.maximum(t, 0.0)
    z2_ref[...] = dinv * jnp.dot(out1, w2_ref[...],
                                 preferred_element_type=jnp.float32)


def _tc_softmax_body(s2_ref, z2_ref, degT_ref, b2_ref, out_ref):
    dinv = _dinv_from(degT_ref)
    u = dinv * (s2_ref[0] + s2_ref[1] + z2_ref[...]) + b2_ref[...]
    cols = lax.broadcasted_iota(jnp.int32, (1, CP), 1)
    cmask = cols < C
    u = jnp.where(cmask, u, -jnp.inf)
    m = jnp.max(u, axis=1, keepdims=True)
    e = jnp.where(cmask, jnp.exp(u - m), 0.0)
    out_ref[...] = e / jnp.sum(e, axis=1, keepdims=True)


def kernel(x, edge_index, W1, b1, W2, b2):
    src = edge_index[0]
    dst = edge_index[1]
    npad_e = E_PAD - E
    fill = N + (jnp.arange(npad_e, dtype=jnp.int32) % N_DUMMY_ROWS)
    src3 = jnp.concatenate([src, fill]).reshape(NW, CHUNKS, 128)
    dst3 = jnp.concatenate([dst, fill]).reshape(NW, CHUNKS, 128)
    w2p = jnp.zeros((H, CP), jnp.float32).at[:, :C].set(W2)
    b1r = b1.reshape(1, H)
    b2r = jnp.zeros((1, CP), jnp.float32).at[0, :C].set(b2)

    deg = _sc_degree(dst3)
    degT = deg.T

    z1 = pl.pallas_call(
        _tc_layer1_body,
        out_shape=jax.ShapeDtypeStruct((N_PAD, H), jnp.float32),
    )(x, W1, degT)

    s1 = _sc_prop16(z1, src3, dst3)

    z2 = pl.pallas_call(
        _tc_layer2_body,
        out_shape=jax.ShapeDtypeStruct((N_PAD, CP), jnp.float32),
    )(s1, z1, degT, b1r, w2p)

    s2 = _sc_prop48(z2, src3, dst3)

    p = pl.pallas_call(
        _tc_softmax_body,
        out_shape=jax.ShapeDtypeStruct((N_PAD, CP), jnp.float32),
    )(s2, z2, degT, b2r)

    return p[:N, :C]


# speedup vs baseline: 54.6787x; 1.0196x over previous
import functools

import jax
import jax.numpy as jnp
from jax import lax
from jax.experimental import pallas as pl
from jax.experimental.pallas import tpu as pltpu
from jax.experimental.pallas import tpu_sc as plsc

N = 10000
E = 320000
D_IN = 128
H = 16
C = 40
CP = 48

NW = 32
N_PAD = 10240
ROWS_PER_TILE = N_PAD // 16
E_PAD = 327680
CHUNKS = E_PAD // (NW * 128)
N_DUMMY_ROWS = 128

_MESH = plsc.VectorSubcoreMesh(core_axis_name="c", subcore_axis_name="s")
_SC_PARAMS = pltpu.CompilerParams(use_tc_tiling_on_sc=False)


@functools.partial(
    pl.kernel,
    mesh=_MESH,
    out_type=jax.ShapeDtypeStruct((2, N_PAD), jnp.float32),
    compiler_params=_SC_PARAMS,
    scratch_types=[
        pltpu.VMEM((CHUNKS, 128), jnp.int32),
        pltpu.VMEM((128,), jnp.float32),
        pltpu.VMEM((ROWS_PER_TILE,), jnp.float32),
        pltpu.VMEM_SHARED((N_PAD,), jnp.float32),
    ],
)
def _sc_degree(dst_hbm, deg_out, idx_v, ones_v, stage_v, acc_sh):
    c = lax.axis_index("c")
    s = lax.axis_index("s")
    wid = s * 2 + c

    def zstage(i, carry):
        stage_v[pl.ds(i * 16, 16)] = jnp.zeros((16,), jnp.float32)
        return carry

    lax.fori_loop(0, ROWS_PER_TILE // 16, zstage, 0)
    pltpu.sync_copy(stage_v, acc_sh.at[pl.ds(s * ROWS_PER_TILE, ROWS_PER_TILE)])
    for k in range(8):
        ones_v[pl.ds(k * 16, 16)] = jnp.ones((16,), jnp.float32)
    plsc.subcore_barrier()

    pltpu.sync_copy(dst_hbm.at[wid], idx_v)

    def body(j, carry):
        pltpu.sync_copy(ones_v, acc_sh.at[idx_v.at[j]], add=True)
        return carry

    lax.fori_loop(0, CHUNKS, body, 0)
    plsc.subcore_barrier()

    pltpu.sync_copy(acc_sh.at[pl.ds(s * ROWS_PER_TILE, ROWS_PER_TILE)], stage_v)
    pltpu.sync_copy(stage_v, deg_out.at[c, pl.ds(s * ROWS_PER_TILE, ROWS_PER_TILE)])


NBUF = 10
GA = 5


def _make_sc_prop(width):
    @functools.partial(
        pl.kernel,
        mesh=_MESH,
        out_type=jax.ShapeDtypeStruct((2, N_PAD, width), jnp.float32),
        compiler_params=_SC_PARAMS,
        scratch_types=[
            pltpu.VMEM((CHUNKS, 128), jnp.int32),
            pltpu.VMEM((CHUNKS, 128), jnp.int32),
            pltpu.VMEM((NBUF, 128, width), jnp.float32),
            pltpu.VMEM((128, width), jnp.float32),
            pltpu.VMEM_SHARED((N_PAD, width), jnp.float32),
            pltpu.SemaphoreType.DMA((NBUF,)),
            pltpu.SemaphoreType.DMA((NBUF,)),
        ],
    )
    def _sc_prop(z_hbm, src_hbm, dst_hbm, out_hbm, si_v, di_v, gbufs, zbuf,
                 acc_sh, semg, sems):
        c = lax.axis_index("c")
        s = lax.axis_index("s")
        wid = s * 2 + c

        ld_si = pltpu.async_copy(src_hbm.at[wid], si_v, semg.at[0])
        ld_di = pltpu.async_copy(dst_hbm.at[wid], di_v, semg.at[1])

        def zb(i, carry):
            for k in range(width // 16):
                zbuf[i, pl.ds(k * 16, 16)] = jnp.zeros((16,), jnp.float32)
            return carry

        lax.fori_loop(0, 128, zb, 0)

        def zacc(i, carry):
            pltpu.sync_copy(
                zbuf, acc_sh.at[pl.ds(s * ROWS_PER_TILE + i * 128, 128)])
            return carry

        lax.fori_loop(0, ROWS_PER_TILE // 128, zacc, 0)
        ld_si.wait()
        ld_di.wait()
        plsc.subcore_barrier()

        for b in range(GA):
            pltpu.async_copy(z_hbm.at[si_v.at[b]], gbufs.at[b], semg.at[b])

        def group(g, carry):
            for b in range(NBUF):
                j = g * NBUF + b
                pltpu.make_async_copy(
                    z_hbm.at[si_v.at[j]], gbufs.at[b], semg.at[b]).wait()
                pltpu.async_copy(
                    gbufs.at[b], acc_sh.at[di_v.at[j]], sems.at[b], add=True)
                bc = (b + GA) % NBUF

                def refill():
                    pltpu.make_async_copy(
                        gbufs.at[bc], acc_sh.at[di_v.at[j]],
                        sems.at[bc]).wait()
                    pltpu.async_copy(
                        z_hbm.at[si_v.at[j + GA]], gbufs.at[bc], semg.at[bc])

                if b < GA:
                    @pl.when(g > 0)
                    def _():
                        refill()

                    @pl.when(g == 0)
                    def _():
                        pltpu.async_copy(z_hbm.at[si_v.at[j + GA]],
                                         gbufs.at[bc], semg.at[bc])
                else:
                    @pl.when(g < CHUNKS // NBUF - 1)
                    def _():
                        refill()
            return carry

        lax.fori_loop(0, CHUNKS // NBUF, group, 0)
        for b in range(NBUF):
            pltpu.make_async_copy(
                gbufs.at[b], acc_sh.at[di_v.at[CHUNKS - 1]], sems.at[b]).wait()
        plsc.subcore_barrier()

        nwb = ROWS_PER_TILE // 128
        for i in range(nwb):
            base = s * ROWS_PER_TILE + i * 128
            pltpu.async_copy(acc_sh.at[pl.ds(base, 128)], gbufs.at[i],
                             semg.at[i])
        for i in range(nwb):
            base = s * ROWS_PER_TILE + i * 128
            pltpu.make_async_copy(acc_sh.at[pl.ds(base, 128)], gbufs.at[i],
                                  semg.at[i]).wait()
            pltpu.async_copy(gbufs.at[i], out_hbm.at[c, pl.ds(base, 128)],
                             sems.at[i])
        for i in range(nwb):
            base = s * ROWS_PER_TILE + i * 128
            pltpu.make_async_copy(gbufs.at[i], out_hbm.at[c, pl.ds(base, 128)],
                                  sems.at[i]).wait()

    return _sc_prop


_sc_prop16 = _make_sc_prop(H)
_sc_prop48 = _make_sc_prop(CP)


def _dinv_from(degT_ref):
    d = degT_ref[:, 0:1] + degT_ref[:, 1:2] + 1.0
    rows = lax.broadcasted_iota(jnp.int32, (N_PAD, 1), 0)
    return jnp.where(rows < N, lax.rsqrt(d), 0.0)


def _tc_layer1_body(x_ref, w1_ref, degT_ref, z1_ref):
    h = jnp.dot(x_ref[...], w1_ref[...], preferred_element_type=jnp.float32)
    dinv = _dinv_from(degT_ref)
    z1_ref[pl.ds(0, N), :] = dinv[:N, :] * h
    z1_ref[pl.ds(N, N_PAD - N), :] = jnp.zeros((N_PAD - N, H), jnp.float32)


def _tc_layer2_body(s1_ref, z1_ref, degT_ref, b1_ref, w2_ref, z2_ref):
    dinv = _dinv_from(degT_ref)
    t = dinv * (s1_ref[0] + s1_ref[1] + z1_ref[...]) + b1_ref[...]
    out1 = jnp.maximum(t, 0.0)
    z2_ref[...] = dinv * jnp.dot(out1, w2_ref[...],
                                 preferred_element_type=jnp.float32)


def _tc_softmax_body(s2_ref, z2_ref, degT_ref, b2_ref, out_ref):
    dinv = _dinv_from(degT_ref)
    u = dinv * (s2_ref[0] + s2_ref[1] + z2_ref[...]) + b2_ref[...]
    cols = lax.broadcasted_iota(jnp.int32, (1, CP), 1)
    cmask = cols < C
    u = jnp.where(cmask, u, -jnp.inf)
    m = jnp.max(u, axis=1, keepdims=True)
    e = jnp.where(cmask, jnp.exp(u - m), 0.0)
    out_ref[...] = e / jnp.sum(e, axis=1, keepdims=True)


def kernel(x, edge_index, W1, b1, W2, b2):
    src = edge_index[0]
    dst = edge_index[1]
    npad_e = E_PAD - E
    fill = N + (jnp.arange(npad_e, dtype=jnp.int32) % N_DUMMY_ROWS)
    src3 = jnp.concatenate([src, fill]).reshape(NW, CHUNKS, 128)
    dst3 = jnp.concatenate([dst, fill]).reshape(NW, CHUNKS, 128)
    w2p = jnp.zeros((H, CP), jnp.float32).at[:, :C].set(W2)
    b1r = b1.reshape(1, H)
    b2r = jnp.zeros((1, CP), jnp.float32).at[0, :C].set(b2)

    deg = _sc_degree(dst3)
    degT = deg.T

    z1 = pl.pallas_call(
        _tc_layer1_body,
        out_shape=jax.ShapeDtypeStruct((N_PAD, H), jnp.float32),
    )(x, W1, degT)

    s1 = _sc_prop16(z1, src3, dst3)

    z2 = pl.pallas_call(
        _tc_layer2_body,
        out_shape=jax.ShapeDtypeStruct((N_PAD, CP), jnp.float32),
    )(s1, z1, degT, b1r, w2p)

    s2 = _sc_prop48(z2, src3, dst3)

    p = pl.pallas_call(
        _tc_softmax_body,
        out_shape=jax.ShapeDtypeStruct((N_PAD, CP), jnp.float32),
    )(s2, z2, degT, b2r)

    return p[:N, :C]


# speedup vs baseline: 55.5598x; 1.0161x over previous
import functools

import jax
import jax.numpy as jnp
from jax import lax
from jax.experimental import pallas as pl
from jax.experimental.pallas import tpu as pltpu
from jax.experimental.pallas import tpu_sc as plsc

N = 10000
E = 320000
D_IN = 128
H = 16
C = 40
CP = 48

NW = 32
N_PAD = 10240
ROWS_PER_TILE = N_PAD // 16
E_PAD = 327680
CHUNKS = E_PAD // (NW * 128)
N_DUMMY_ROWS = 128

_MESH = plsc.VectorSubcoreMesh(core_axis_name="c", subcore_axis_name="s")
_SC_PARAMS = pltpu.CompilerParams(use_tc_tiling_on_sc=False)


@functools.partial(
    pl.kernel,
    mesh=_MESH,
    out_type=jax.ShapeDtypeStruct((2, N_PAD), jnp.float32),
    compiler_params=_SC_PARAMS,
    scratch_types=[
        pltpu.VMEM((CHUNKS, 128), jnp.int32),
        pltpu.VMEM((128,), jnp.float32),
        pltpu.VMEM((ROWS_PER_TILE,), jnp.float32),
        pltpu.VMEM_SHARED((N_PAD,), jnp.float32),
    ],
)
def _sc_degree(dst_hbm, deg_out, idx_v, ones_v, stage_v, acc_sh):
    c = lax.axis_index("c")
    s = lax.axis_index("s")
    wid = s * 2 + c

    def zstage(i, carry):
        stage_v[pl.ds(i * 16, 16)] = jnp.zeros((16,), jnp.float32)
        return carry

    lax.fori_loop(0, ROWS_PER_TILE // 16, zstage, 0)
    pltpu.sync_copy(stage_v, acc_sh.at[pl.ds(s * ROWS_PER_TILE, ROWS_PER_TILE)])
    for k in range(8):
        ones_v[pl.ds(k * 16, 16)] = jnp.ones((16,), jnp.float32)
    plsc.subcore_barrier()

    pltpu.sync_copy(dst_hbm.at[wid], idx_v)

    def body(j, carry):
        pltpu.sync_copy(ones_v, acc_sh.at[idx_v.at[j]], add=True)
        return carry

    lax.fori_loop(0, CHUNKS, body, 0)
    plsc.subcore_barrier()

    pltpu.sync_copy(acc_sh.at[pl.ds(s * ROWS_PER_TILE, ROWS_PER_TILE)], stage_v)
    pltpu.sync_copy(stage_v, deg_out.at[c, pl.ds(s * ROWS_PER_TILE, ROWS_PER_TILE)])


NBUF = 10
GA = 5


def _make_sc_prop(width, stage_z):
    @functools.partial(
        pl.kernel,
        mesh=_MESH,
        out_type=jax.ShapeDtypeStruct((2, N_PAD, width), jnp.float32),
        compiler_params=_SC_PARAMS,
        scratch_types=[
            pltpu.VMEM((CHUNKS, 128), jnp.int32),
            pltpu.VMEM((CHUNKS, 128), jnp.int32),
            pltpu.VMEM((NBUF, 128, width), jnp.float32),
            pltpu.VMEM((128, width), jnp.float32),
            pltpu.VMEM_SHARED((N_PAD, width), jnp.float32),
            pltpu.VMEM_SHARED((N_PAD if stage_z else 1, width), jnp.float32),
            pltpu.SemaphoreType.DMA((NBUF,)),
            pltpu.SemaphoreType.DMA((NBUF,)),
        ],
    )
    def _sc_prop(z_hbm, src_hbm, dst_hbm, out_hbm, si_v, di_v, gbufs, zbuf,
                 acc_sh, z_sh, semg, sems):
        c = lax.axis_index("c")
        s = lax.axis_index("s")
        wid = s * 2 + c

        ld_si = pltpu.async_copy(src_hbm.at[wid], si_v, semg.at[0])
        ld_di = pltpu.async_copy(dst_hbm.at[wid], di_v, semg.at[1])

        def zb(i, carry):
            for k in range(width // 16):
                zbuf[i, pl.ds(k * 16, 16)] = jnp.zeros((16,), jnp.float32)
            return carry

        lax.fori_loop(0, 128, zb, 0)

        def zacc(i, carry):
            pltpu.sync_copy(
                zbuf, acc_sh.at[pl.ds(s * ROWS_PER_TILE + i * 128, 128)])
            return carry

        lax.fori_loop(0, ROWS_PER_TILE // 128, zacc, 0)
        if stage_z:
            for i in range(ROWS_PER_TILE // 128):
                base = s * ROWS_PER_TILE + i * 128
                pltpu.async_copy(z_hbm.at[pl.ds(base, 128)], gbufs.at[i],
                                 sems.at[i])
            for i in range(ROWS_PER_TILE // 128):
                base = s * ROWS_PER_TILE + i * 128
                pltpu.make_async_copy(z_hbm.at[pl.ds(base, 128)], gbufs.at[i],
                                      sems.at[i]).wait()
                pltpu.sync_copy(gbufs.at[i], z_sh.at[pl.ds(base, 128)])
        ld_si.wait()
        ld_di.wait()
        plsc.subcore_barrier()
        z_src = z_sh if stage_z else z_hbm

        for b in range(GA):
            pltpu.async_copy(z_src.at[si_v.at[b]], gbufs.at[b], semg.at[b])

        def group(g, carry):
            for b in range(NBUF):
                j = g * NBUF + b
                pltpu.make_async_copy(
                    z_src.at[si_v.at[j]], gbufs.at[b], semg.at[b]).wait()
                pltpu.async_copy(
                    gbufs.at[b], acc_sh.at[di_v.at[j]], sems.at[b], add=True)
                bc = (b + GA) % NBUF

                def refill():
                    pltpu.make_async_copy(
                        gbufs.at[bc], acc_sh.at[di_v.at[j]],
                        sems.at[bc]).wait()
                    pltpu.async_copy(
                        z_src.at[si_v.at[j + GA]], gbufs.at[bc], semg.at[bc])

                if b < GA:
                    @pl.when(g > 0)
                    def _():
                        refill()

                    @pl.when(g == 0)
                    def _():
                        pltpu.async_copy(z_src.at[si_v.at[j + GA]],
                                         gbufs.at[bc], semg.at[bc])
                else:
                    @pl.when(g < CHUNKS // NBUF - 1)
                    def _():
                        refill()
            return carry

        lax.fori_loop(0, CHUNKS // NBUF, group, 0)
        for b in range(NBUF):
            pltpu.make_async_copy(
                gbufs.at[b], acc_sh.at[di_v.at[CHUNKS - 1]], sems.at[b]).wait()
        plsc.subcore_barrier()

        nwb = ROWS_PER_TILE // 128
        for i in range(nwb):
            base = s * ROWS_PER_TILE + i * 128
            pltpu.async_copy(acc_sh.at[pl.ds(base, 128)], gbufs.at[i],
                             semg.at[i])
        for i in range(nwb):
            base = s * ROWS_PER_TILE + i * 128
            pltpu.make_async_copy(acc_sh.at[pl.ds(base, 128)], gbufs.at[i],
                                  semg.at[i]).wait()
            pltpu.async_copy(gbufs.at[i], out_hbm.at[c, pl.ds(base, 128)],
                             sems.at[i])
        for i in range(nwb):
            base = s * ROWS_PER_TILE + i * 128
            pltpu.make_async_copy(gbufs.at[i], out_hbm.at[c, pl.ds(base, 128)],
                                  sems.at[i]).wait()

    return _sc_prop


_sc_prop16 = _make_sc_prop(H, stage_z=True)
_sc_prop48 = _make_sc_prop(CP, stage_z=False)


def _dinv_from(degT_ref):
    d = degT_ref[:, 0:1] + degT_ref[:, 1:2] + 1.0
    rows = lax.broadcasted_iota(jnp.int32, (N_PAD, 1), 0)
    return jnp.where(rows < N, lax.rsqrt(d), 0.0)


def _tc_layer1_body(x_ref, w1_ref, degT_ref, z1_ref):
    h = jnp.dot(x_ref[...], w1_ref[...], preferred_element_type=jnp.float32)
    dinv = _dinv_from(degT_ref)
    z1_ref[pl.ds(0, N), :] = dinv[:N, :] * h
    z1_ref[pl.ds(N, N_PAD - N), :] = jnp.zeros((N_PAD - N, H), jnp.float32)


def _tc_layer2_body(s1_ref, z1_ref, degT_ref, b1_ref, w2_ref, z2_ref):
    dinv = _dinv_from(degT_ref)
    t = dinv * (s1_ref[0] + s1_ref[1] + z1_ref[...]) + b1_ref[...]
    out1 = jnp.maximum(t, 0.0)
    z2_ref[...] = dinv * jnp.dot(out1, w2_ref[...],
                                 preferred_element_type=jnp.float32)


def _tc_softmax_body(s2_ref, z2_ref, degT_ref, b2_ref, out_ref):
    dinv = _dinv_from(degT_ref)
    u = dinv * (s2_ref[0] + s2_ref[1] + z2_ref[...]) + b2_ref[...]
    cols = lax.broadcasted_iota(jnp.int32, (1, CP), 1)
    cmask = cols < C
    u = jnp.where(cmask, u, -jnp.inf)
    m = jnp.max(u, axis=1, keepdims=True)
    e = jnp.where(cmask, jnp.exp(u - m), 0.0)
    out_ref[...] = e / jnp.sum(e, axis=1, keepdims=True)


def kernel(x, edge_index, W1, b1, W2, b2):
    src = edge_index[0]
    dst = edge_index[1]
    npad_e = E_PAD - E
    fill = N + (jnp.arange(npad_e, dtype=jnp.int32) % N_DUMMY_ROWS)
    src3 = jnp.concatenate([src, fill]).reshape(NW, CHUNKS, 128)
    dst3 = jnp.concatenate([dst, fill]).reshape(NW, CHUNKS, 128)
    w2p = jnp.zeros((H, CP), jnp.float32).at[:, :C].set(W2)
    b1r = b1.reshape(1, H)
    b2r = jnp.zeros((1, CP), jnp.float32).at[0, :C].set(b2)

    deg = _sc_degree(dst3)
    degT = deg.T

    z1 = pl.pallas_call(
        _tc_layer1_body,
        out_shape=jax.ShapeDtypeStruct((N_PAD, H), jnp.float32),
    )(x, W1, degT)

    s1 = _sc_prop16(z1, src3, dst3)

    z2 = pl.pallas_call(
        _tc_layer2_body,
        out_shape=jax.ShapeDtypeStruct((N_PAD, CP), jnp.float32),
    )(s1, z1, degT, b1r, w2p)

    s2 = _sc_prop48(z2, src3, dst3)

    p = pl.pallas_call(
        _tc_softmax_body,
        out_shape=jax.ShapeDtypeStruct((N_PAD, CP), jnp.float32),
    )(s2, z2, degT, b2r)

    return p[:N, :C]


# speedup vs baseline: 55.5796x; 1.0004x over previous
import functools

import jax
import jax.numpy as jnp
from jax import lax
from jax.experimental import pallas as pl
from jax.experimental.pallas import tpu as pltpu
from jax.experimental.pallas import tpu_sc as plsc

N = 10000
E = 320000
D_IN = 128
H = 16
C = 40
CP = 48

NW = 32
N_PAD = 10240
ROWS_PER_TILE = N_PAD // 16
E_PAD = 327680
CHUNKS = E_PAD // (NW * 128)
N_DUMMY_ROWS = 128

_MESH = plsc.VectorSubcoreMesh(core_axis_name="c", subcore_axis_name="s")
_SC_PARAMS = pltpu.CompilerParams(use_tc_tiling_on_sc=False,
                                  skip_device_barrier=True)


@functools.partial(
    pl.kernel,
    mesh=_MESH,
    out_type=jax.ShapeDtypeStruct((2, N_PAD), jnp.float32),
    compiler_params=_SC_PARAMS,
    scratch_types=[
        pltpu.VMEM((CHUNKS, 128), jnp.int32),
        pltpu.VMEM((128,), jnp.float32),
        pltpu.VMEM((ROWS_PER_TILE,), jnp.float32),
        pltpu.VMEM_SHARED((N_PAD,), jnp.float32),
    ],
)
def _sc_degree(dst_hbm, deg_out, idx_v, ones_v, stage_v, acc_sh):
    c = lax.axis_index("c")
    s = lax.axis_index("s")
    wid = s * 2 + c

    def zstage(i, carry):
        stage_v[pl.ds(i * 16, 16)] = jnp.zeros((16,), jnp.float32)
        return carry

    lax.fori_loop(0, ROWS_PER_TILE // 16, zstage, 0)
    pltpu.sync_copy(stage_v, acc_sh.at[pl.ds(s * ROWS_PER_TILE, ROWS_PER_TILE)])
    for k in range(8):
        ones_v[pl.ds(k * 16, 16)] = jnp.ones((16,), jnp.float32)
    plsc.subcore_barrier()

    pltpu.sync_copy(dst_hbm.at[wid], idx_v)

    def body(j, carry):
        pltpu.sync_copy(ones_v, acc_sh.at[idx_v.at[j]], add=True)
        return carry

    lax.fori_loop(0, CHUNKS, body, 0)
    plsc.subcore_barrier()

    pltpu.sync_copy(acc_sh.at[pl.ds(s * ROWS_PER_TILE, ROWS_PER_TILE)], stage_v)
    pltpu.sync_copy(stage_v, deg_out.at[c, pl.ds(s * ROWS_PER_TILE, ROWS_PER_TILE)])


NBUF = 10
GA = 5


def _make_sc_prop(width, stage_z):
    @functools.partial(
        pl.kernel,
        mesh=_MESH,
        out_type=jax.ShapeDtypeStruct((2, N_PAD, width), jnp.float32),
        compiler_params=_SC_PARAMS,
        scratch_types=[
            pltpu.VMEM((CHUNKS, 128), jnp.int32),
            pltpu.VMEM((CHUNKS, 128), jnp.int32),
            pltpu.VMEM((NBUF, 128, width), jnp.float32),
            pltpu.VMEM((128, width), jnp.float32),
            pltpu.VMEM_SHARED((N_PAD, width), jnp.float32),
            pltpu.VMEM_SHARED((N_PAD if stage_z else 1, width), jnp.float32),
            pltpu.SemaphoreType.DMA((NBUF,)),
            pltpu.SemaphoreType.DMA((NBUF,)),
        ],
    )
    def _sc_prop(z_hbm, src_hbm, dst_hbm, out_hbm, si_v, di_v, gbufs, zbuf,
                 acc_sh, z_sh, semg, sems):
        c = lax.axis_index("c")
        s = lax.axis_index("s")
        wid = s * 2 + c

        ld_si = pltpu.async_copy(src_hbm.at[wid], si_v, semg.at[0])
        ld_di = pltpu.async_copy(dst_hbm.at[wid], di_v, semg.at[1])

        def zb(i, carry):
            for k in range(width // 16):
                zbuf[i, pl.ds(k * 16, 16)] = jnp.zeros((16,), jnp.float32)
            return carry

        lax.fori_loop(0, 128, zb, 0)

        def zacc(i, carry):
            pltpu.sync_copy(
                zbuf, acc_sh.at[pl.ds(s * ROWS_PER_TILE + i * 128, 128)])
            return carry

        lax.fori_loop(0, ROWS_PER_TILE // 128, zacc, 0)
        if stage_z:
            for i in range(ROWS_PER_TILE // 128):
                base = s * ROWS_PER_TILE + i * 128
                pltpu.async_copy(z_hbm.at[pl.ds(base, 128)], gbufs.at[i],
                                 sems.at[i])
            for i in range(ROWS_PER_TILE // 128):
                base = s * ROWS_PER_TILE + i * 128
                pltpu.make_async_copy(z_hbm.at[pl.ds(base, 128)], gbufs.at[i],
                                      sems.at[i]).wait()
                pltpu.sync_copy(gbufs.at[i], z_sh.at[pl.ds(base, 128)])
        ld_si.wait()
        ld_di.wait()
        plsc.subcore_barrier()
        z_src = z_sh if stage_z else z_hbm

        for b in range(GA):
            pltpu.async_copy(z_src.at[si_v.at[b]], gbufs.at[b], semg.at[b])

        def group(g, carry):
            for b in range(NBUF):
                j = g * NBUF + b
                pltpu.make_async_copy(
                    z_src.at[si_v.at[j]], gbufs.at[b], semg.at[b]).wait()
                pltpu.async_copy(
                    gbufs.at[b], acc_sh.at[di_v.at[j]], sems.at[b], add=True)
                bc = (b + GA) % NBUF

                def refill():
                    pltpu.make_async_copy(
                        gbufs.at[bc], acc_sh.at[di_v.at[j]],
                        sems.at[bc]).wait()
                    pltpu.async_copy(
                        z_src.at[si_v.at[j + GA]], gbufs.at[bc], semg.at[bc])

                if b < GA:
                    @pl.when(g > 0)
                    def _():
                        refill()

                    @pl.when(g == 0)
                    def _():
                        pltpu.async_copy(z_src.at[si_v.at[j + GA]],
                                         gbufs.at[bc], semg.at[bc])
                else:
                    @pl.when(g < CHUNKS // NBUF - 1)
                    def _():
                        refill()
            return carry

        lax.fori_loop(0, CHUNKS // NBUF, group, 0)
        for b in range(NBUF):
            pltpu.make_async_copy(
                gbufs.at[b], acc_sh.at[di_v.at[CHUNKS - 1]], sems.at[b]).wait()
        plsc.subcore_barrier()

        nwb = ROWS_PER_TILE // 128
        for i in range(nwb):
            base = s * ROWS_PER_TILE + i * 128
            pltpu.async_copy(acc_sh.at[pl.ds(base, 128)], gbufs.at[i],
                             semg.at[i])
        for i in range(nwb):
            base = s * ROWS_PER_TILE + i * 128
            pltpu.make_async_copy(acc_sh.at[pl.ds(base, 128)], gbufs.at[i],
                                  semg.at[i]).wait()
            pltpu.async_copy(gbufs.at[i], out_hbm.at[c, pl.ds(base, 128)],
                             sems.at[i])
        for i in range(nwb):
            base = s * ROWS_PER_TILE + i * 128
            pltpu.make_async_copy(gbufs.at[i], out_hbm.at[c, pl.ds(base, 128)],
                                  sems.at[i]).wait()

    return _sc_prop


_sc_prop16 = _make_sc_prop(H, stage_z=True)
_sc_prop48 = _make_sc_prop(CP, stage_z=False)


def _dinv_from(degT_ref):
    d = degT_ref[:, 0:1] + degT_ref[:, 1:2] + 1.0
    rows = lax.broadcasted_iota(jnp.int32, (N_PAD, 1), 0)
    return jnp.where(rows < N, lax.rsqrt(d), 0.0)


def _tc_layer1_body(x_ref, w1_ref, degT_ref, z1_ref):
    h = jnp.dot(x_ref[...], w1_ref[...], preferred_element_type=jnp.float32)
    dinv = _dinv_from(degT_ref)
    z1_ref[pl.ds(0, N), :] = dinv[:N, :] * h
    z1_ref[pl.ds(N, N_PAD - N), :] = jnp.zeros((N_PAD - N, H), jnp.float32)


def _tc_layer2_body(s1_ref, z1_ref, degT_ref, b1_ref, w2_ref, z2_ref):
    dinv = _dinv_from(degT_ref)
    t = dinv * (s1_ref[0] + s1_ref[1] + z1_ref[...]) + b1_ref[...]
    out1 = jnp.maximum(t, 0.0)
    z2_ref[...] = dinv * jnp.dot(out1, w2_ref[...],
                                 preferred_element_type=jnp.float32)


def _tc_softmax_body(s2_ref, z2_ref, degT_ref, b2_ref, out_ref):
    dinv = _dinv_from(degT_ref)
    u = dinv * (s2_ref[0] + s2_ref[1] + z2_ref[...]) + b2_ref[...]
    cols = lax.broadcasted_iota(jnp.int32, (1, CP), 1)
    cmask = cols < C
    u = jnp.where(cmask, u, -jnp.inf)
    m = jnp.max(u, axis=1, keepdims=True)
    e = jnp.where(cmask, jnp.exp(u - m), 0.0)
    out_ref[...] = e / jnp.sum(e, axis=1, keepdims=True)


def kernel(x, edge_index, W1, b1, W2, b2):
    src = edge_index[0]
    dst = edge_index[1]
    npad_e = E_PAD - E
    fill = N + (jnp.arange(npad_e, dtype=jnp.int32) % N_DUMMY_ROWS)
    src3 = jnp.concatenate([src, fill]).reshape(NW, CHUNKS, 128)
    dst3 = jnp.concatenate([dst, fill]).reshape(NW, CHUNKS, 128)
    w2p = jnp.zeros((H, CP), jnp.float32).at[:, :C].set(W2)
    b1r = b1.reshape(1, H)
    b2r = jnp.zeros((1, CP), jnp.float32).at[0, :C].set(b2)

    deg = _sc_degree(dst3)
    degT = deg.T

    z1 = pl.pallas_call(
        _tc_layer1_body,
        out_shape=jax.ShapeDtypeStruct((N_PAD, H), jnp.float32),
    )(x, W1, degT)

    s1 = _sc_prop16(z1, src3, dst3)

    z2 = pl.pallas_call(
        _tc_layer2_body,
        out_shape=jax.ShapeDtypeStruct((N_PAD, CP), jnp.float32),
    )(s1, z1, degT, b1r, w2p)

    s2 = _sc_prop48(z2, src3, dst3)

    p = pl.pallas_call(
        _tc_softmax_body,
        out_shape=jax.ShapeDtypeStruct((N_PAD, CP), jnp.float32),
    )(s2, z2, degT, b2r)

    return p[:N, :C]


# speedup vs baseline: 59.8074x; 1.0761x over previous
import functools

import jax
import jax.numpy as jnp
from jax import lax
from jax.experimental import pallas as pl
from jax.experimental.pallas import tpu as pltpu
from jax.experimental.pallas import tpu_sc as plsc

N = 10000
E = 320000
D_IN = 128
H = 16
C = 40
CP = 48

NW = 32
N_PAD = 10240
RPT = N_PAD // 16
E_PAD = 327680
CHUNKS = E_PAD // (NW * 128)
N_DUMMY_ROWS = 128

_MESH = plsc.VectorSubcoreMesh(core_axis_name="c", subcore_axis_name="s")
_SC_PARAMS = pltpu.CompilerParams(use_tc_tiling_on_sc=False)
_SC_PARAMS_NL = pltpu.CompilerParams(use_tc_tiling_on_sc=False,
                                     needs_layout_passes=False)

NBUF = 10
GA = 5


@functools.partial(
    pl.kernel,
    mesh=_MESH,
    out_type=jax.ShapeDtypeStruct((2, N_PAD), jnp.float32),
    compiler_params=_SC_PARAMS,
    scratch_types=[
        pltpu.VMEM((CHUNKS, 128), jnp.int32),
        pltpu.VMEM((128,), jnp.float32),
        pltpu.VMEM((RPT,), jnp.float32),
        pltpu.VMEM_SHARED((N_PAD,), jnp.float32),
    ],
)
def _sc_degree(dst_hbm, deg_out, idx_v, ones_v, stage_v, acc_sh):
    c = lax.axis_index("c")
    s = lax.axis_index("s")
    wid = s * 2 + c

    def zstage(i, carry):
        stage_v[pl.ds(i * 16, 16)] = jnp.zeros((16,), jnp.float32)
        return carry

    lax.fori_loop(0, RPT // 16, zstage, 0)
    pltpu.sync_copy(stage_v, acc_sh.at[pl.ds(s * RPT, RPT)])
    for k in range(8):
        ones_v[pl.ds(k * 16, 16)] = jnp.ones((16,), jnp.float32)
    plsc.subcore_barrier()

    pltpu.sync_copy(dst_hbm.at[wid], idx_v)

    def body(j, carry):
        pltpu.sync_copy(ones_v, acc_sh.at[idx_v.at[j]], add=True)
        return carry

    lax.fori_loop(0, CHUNKS, body, 0)
    plsc.subcore_barrier()

    pltpu.sync_copy(acc_sh.at[pl.ds(s * RPT, RPT)], stage_v)
    pltpu.sync_copy(stage_v, deg_out.at[c, pl.ds(s * RPT, RPT)])


def _sc_bcast(v, l):
    return lax.gather(
        v, jnp.full((16, 1), l, jnp.int32),
        dimension_numbers=lax.GatherDimensionNumbers(
            offset_dims=(), collapsed_slice_dims=(0,), start_index_map=(0,)),
        slice_sizes=(1,), mode=lax.GatherScatterMode.PROMISE_IN_BOUNDS)


def _sc_rsqrt(d):
    ti = plsc.bitcast(d, jnp.int32)
    ti = jnp.full((16,), 0x5F3759DF, jnp.int32) - (ti >> 1)
    y = plsc.bitcast(ti, jnp.float32)
    for _ in range(3):
        y = y * (1.5 - 0.5 * d * y * y)
    return y


def _pipeline_prop(z_src, si_v, di_v, gbufs, acc_sh, semg, sems):
    for b in range(GA):
        pltpu.async_copy(z_src.at[si_v.at[b]], gbufs.at[b], semg.at[b])

    def group(g, carry):
        for b in range(NBUF):
            j = g * NBUF + b
            pltpu.make_async_copy(
                z_src.at[si_v.at[j]], gbufs.at[b], semg.at[b]).wait()
            pltpu.async_copy(
                gbufs.at[b], acc_sh.at[di_v.at[j]], sems.at[b], add=True)
            bc = (b + GA) % NBUF

            def refill():
                pltpu.make_async_copy(
                    gbufs.at[bc], acc_sh.at[di_v.at[j]], sems.at[bc]).wait()
                pltpu.async_copy(
                    z_src.at[si_v.at[j + GA]], gbufs.at[bc], semg.at[bc])

            if b < GA:
                @pl.when(g > 0)
                def _():
                    refill()

                @pl.when(g == 0)
                def _():
                    pltpu.async_copy(z_src.at[si_v.at[j + GA]],
                                     gbufs.at[bc], semg.at[bc])
            else:
                @pl.when(g < CHUNKS // NBUF - 1)
                def _():
                    refill()
        return carry

    lax.fori_loop(0, CHUNKS // NBUF, group, 0)
    for b in range(NBUF):
        pltpu.make_async_copy(
            gbufs.at[b], acc_sh.at[di_v.at[CHUNKS - 1]], sems.at[b]).wait()


def _writeback(out_hbm, c, s, gbufs, acc_sh, semg, sems):
    nwb = RPT // 128
    for i in range(nwb):
        base = s * RPT + i * 128
        pltpu.async_copy(acc_sh.at[pl.ds(base, 128)], gbufs.at[i], semg.at[i])
    for i in range(nwb):
        base = s * RPT + i * 128
        pltpu.make_async_copy(acc_sh.at[pl.ds(base, 128)], gbufs.at[i],
                              semg.at[i]).wait()
        pltpu.async_copy(gbufs.at[i], out_hbm.at[c, pl.ds(base, 128)],
                         sems.at[i])
    for i in range(nwb):
        base = s * RPT + i * 128
        pltpu.make_async_copy(gbufs.at[i], out_hbm.at[c, pl.ds(base, 128)],
                              sems.at[i]).wait()


@functools.partial(
    pl.kernel,
    mesh=_MESH,
    out_type=jax.ShapeDtypeStruct((2, N_PAD, H), jnp.float32),
    compiler_params=_SC_PARAMS_NL,
    scratch_types=[
        pltpu.VMEM((CHUNKS, 128), jnp.int32),
        pltpu.VMEM((CHUNKS, 128), jnp.int32),
        pltpu.VMEM((NBUF, 128, H), jnp.float32),
        pltpu.VMEM((128, H), jnp.float32),
        pltpu.VMEM((2, RPT), jnp.float32),
        pltpu.VMEM_SHARED((N_PAD, H), jnp.float32),
        pltpu.VMEM_SHARED((N_PAD, H), jnp.float32),
        pltpu.SemaphoreType.DMA((NBUF,)),
        pltpu.SemaphoreType.DMA((NBUF,)),
    ],
)
def _sc_prop1(y1_hbm, deg_hbm, src_hbm, dst_hbm, out_hbm, si_v, di_v, gbufs,
              zbuf, dbuf, acc_sh, z_sh, semg, sems):
    c = lax.axis_index("c")
    s = lax.axis_index("s")
    wid = s * 2 + c

    ld_si = pltpu.async_copy(src_hbm.at[wid], si_v, semg.at[0])
    ld_di = pltpu.async_copy(dst_hbm.at[wid], di_v, semg.at[1])
    ld_d0 = pltpu.async_copy(deg_hbm.at[0, pl.ds(s * RPT, RPT)], dbuf.at[0],
                             semg.at[2])
    ld_d1 = pltpu.async_copy(deg_hbm.at[1, pl.ds(s * RPT, RPT)], dbuf.at[1],
                             semg.at[3])
    for i in range(RPT // 128):
        pltpu.async_copy(y1_hbm.at[pl.ds(s * RPT + i * 128, 128)],
                         gbufs.at[i], sems.at[i])

    def zb(i, carry):
        zbuf[i, :] = jnp.zeros((16,), jnp.float32)
        return carry

    lax.fori_loop(0, 128, zb, 0)

    def zacc(i, carry):
        pltpu.sync_copy(zbuf, acc_sh.at[pl.ds(s * RPT + i * 128, 128)])
        return carry

    lax.fori_loop(0, RPT // 128, zacc, 0)
    ld_d0.wait()
    ld_d1.wait()

    for i in range(RPT // 128):
        pltpu.make_async_copy(y1_hbm.at[pl.ds(s * RPT + i * 128, 128)],
                              gbufs.at[i], sems.at[i]).wait()

        def scale_grp(r, carry):
            base16 = i * 128 + r * 16
            d16 = dbuf[0, pl.ds(base16, 16)] + dbuf[1, pl.ds(base16, 16)] + 1.0
            y = _sc_rsqrt(d16)
            for l in range(16):
                bc = _sc_bcast(y, l)
                gbufs[i, r * 16 + l, :] = gbufs[i, r * 16 + l, :] * bc
            return carry

        lax.fori_loop(0, 8, scale_grp, 0)
        pltpu.sync_copy(gbufs.at[i], z_sh.at[pl.ds(s * RPT + i * 128, 128)])

    ld_si.wait()
    ld_di.wait()
    plsc.subcore_barrier()
    _pipeline_prop(z_sh, si_v, di_v, gbufs, acc_sh, semg, sems)
    plsc.subcore_barrier()
    _writeback(out_hbm, c, s, gbufs, acc_sh, semg, sems)


@functools.partial(
    pl.kernel,
    mesh=_MESH,
    out_type=jax.ShapeDtypeStruct((2, N_PAD, CP), jnp.float32),
    compiler_params=_SC_PARAMS,
    scratch_types=[
        pltpu.VMEM((CHUNKS, 128), jnp.int32),
        pltpu.VMEM((CHUNKS, 128), jnp.int32),
        pltpu.VMEM((NBUF, 128, CP), jnp.float32),
        pltpu.VMEM((128, CP), jnp.float32),
        pltpu.VMEM_SHARED((N_PAD, CP), jnp.float32),
        pltpu.SemaphoreType.DMA((NBUF,)),
        pltpu.SemaphoreType.DMA((NBUF,)),
    ],
)
def _sc_prop2(z_hbm, src_hbm, dst_hbm, out_hbm, si_v, di_v, gbufs, zbuf,
              acc_sh, semg, sems):
    c = lax.axis_index("c")
    s = lax.axis_index("s")
    wid = s * 2 + c

    ld_si = pltpu.async_copy(src_hbm.at[wid], si_v, semg.at[0])
    ld_di = pltpu.async_copy(dst_hbm.at[wid], di_v, semg.at[1])

    def zb(i, carry):
        for k in range(CP // 16):
            zbuf[i, pl.ds(k * 16, 16)] = jnp.zeros((16,), jnp.float32)
        return carry

    lax.fori_loop(0, 128, zb, 0)

    def zacc(i, carry):
        pltpu.sync_copy(zbuf, acc_sh.at[pl.ds(s * RPT + i * 128, 128)])
        return carry

    lax.fori_loop(0, RPT // 128, zacc, 0)
    ld_si.wait()
    ld_di.wait()
    plsc.subcore_barrier()
    _pipeline_prop(z_hbm, si_v, di_v, gbufs, acc_sh, semg, sems)
    plsc.subcore_barrier()
    _writeback(out_hbm, c, s, gbufs, acc_sh, semg, sems)


def _dinv_from(degT_ref):
    d = degT_ref[:, 0:1] + degT_ref[:, 1:2] + 1.0
    rows = lax.broadcasted_iota(jnp.int32, (N_PAD, 1), 0)
    return jnp.where(rows < N, lax.rsqrt(d), 0.0)


def _tc_matmul1_body(x_ref, w1_ref, y1_ref):
    h = jnp.dot(x_ref[...], w1_ref[...], preferred_element_type=jnp.float32)
    y1_ref[pl.ds(0, N), :] = h
    y1_ref[pl.ds(N, N_PAD - N), :] = jnp.zeros((N_PAD - N, H), jnp.float32)


def _tc_layer2_body(s1_ref, y1_ref, degT_ref, b1_ref, w2_ref, z2_ref):
    dinv = _dinv_from(degT_ref)
    z1 = dinv * y1_ref[...]
    t = dinv * (s1_ref[0] + s1_ref[1] + z1) + b1_ref[...]
    out1 = jnp.maximum(t, 0.0)
    z2_ref[...] = dinv * jnp.dot(out1, w2_ref[...],
                                 preferred_element_type=jnp.float32)


def _tc_softmax_body(s2_ref, z2_ref, degT_ref, b2_ref, out_ref):
    dinv = _dinv_from(degT_ref)
    u = dinv * (s2_ref[0] + s2_ref[1] + z2_ref[...]) + b2_ref[...]
    cols = lax.broadcasted_iota(jnp.int32, (1, CP), 1)
    cmask = cols < C
    u = jnp.where(cmask, u, -jnp.inf)
    m = jnp.max(u, axis=1, keepdims=True)
    e = jnp.where(cmask, jnp.exp(u - m), 0.0)
    p = e / jnp.sum(e, axis=1, keepdims=True)
    out_ref[...] = p[:N, :C]


def kernel(x, edge_index, W1, b1, W2, b2):
    src = edge_index[0]
    dst = edge_index[1]
    npad_e = E_PAD - E
    fill = N + (jnp.arange(npad_e, dtype=jnp.int32) % N_DUMMY_ROWS)
    src3 = jnp.concatenate([src, fill]).reshape(NW, CHUNKS, 128)
    dst3 = jnp.concatenate([dst, fill]).reshape(NW, CHUNKS, 128)
    w2p = jnp.zeros((H, CP), jnp.float32).at[:, :C].set(W2)
    b1r = b1.reshape(1, H)
    b2r = jnp.zeros((1, CP), jnp.float32).at[0, :C].set(b2)

    deg = _sc_degree(dst3)
    y1 = pl.pallas_call(
        _tc_matmul1_body,
        out_shape=jax.ShapeDtypeStruct((N_PAD, H), jnp.float32),
    )(x, W1)

    s1 = _sc_prop1(y1, deg, src3, dst3)
    degT = deg.T

    z2 = pl.pallas_call(
        _tc_layer2_body,
        out_shape=jax.ShapeDtypeStruct((N_PAD, CP), jnp.float32),
    )(s1, y1, degT, b1r, w2p)

    s2 = _sc_prop2(z2, src3, dst3)

    return pl.pallas_call(
        _tc_softmax_body,
        out_shape=jax.ShapeDtypeStruct((N, C), jnp.float32),
    )(s2, z2, degT, b2r)


# speedup vs baseline: 60.8671x; 1.0177x over previous
import functools

import jax
import jax.numpy as jnp
from jax import lax
from jax.experimental import pallas as pl
from jax.experimental.pallas import tpu as pltpu
from jax.experimental.pallas import tpu_sc as plsc

N = 10000
E = 320000
D_IN = 128
H = 16
C = 40
CP = 48

NW = 32
N_PAD = 10240
RPT = N_PAD // 16
E_PAD = 327680
CHUNKS = E_PAD // (NW * 128)
N_DUMMY_ROWS = 128

_MESH = plsc.VectorSubcoreMesh(core_axis_name="c", subcore_axis_name="s")
_SC_PARAMS = pltpu.CompilerParams(use_tc_tiling_on_sc=False)
_SC_PARAMS_NL = pltpu.CompilerParams(use_tc_tiling_on_sc=False,
                                     needs_layout_passes=False)

NBUF = 10
GA = 5


@functools.partial(
    pl.kernel,
    mesh=_MESH,
    out_type=jax.ShapeDtypeStruct((2, N_PAD), jnp.float32),
    compiler_params=_SC_PARAMS,
    scratch_types=[
        pltpu.VMEM((CHUNKS, 128), jnp.int32),
        pltpu.VMEM((128,), jnp.float32),
        pltpu.VMEM((RPT,), jnp.float32),
        pltpu.VMEM_SHARED((N_PAD,), jnp.float32),
        pltpu.SemaphoreType.DMA,
    ],
)
def _sc_degree(dst_hbm, deg_out, idx_v, ones_v, stage_v, acc_sh, sem_h):
    c = lax.axis_index("c")
    s = lax.axis_index("s")
    wid = s * 2 + c

    def zstage(i, carry):
        stage_v[pl.ds(i * 16, 16)] = jnp.zeros((16,), jnp.float32)
        return carry

    lax.fori_loop(0, RPT // 16, zstage, 0)
    pltpu.sync_copy(stage_v, acc_sh.at[pl.ds(s * RPT, RPT)])
    for k in range(8):
        ones_v[pl.ds(k * 16, 16)] = jnp.ones((16,), jnp.float32)
    plsc.subcore_barrier()

    pltpu.sync_copy(dst_hbm.at[wid], idx_v)

    def body(j, carry):
        pltpu.async_copy(ones_v, acc_sh.at[idx_v.at[j]], sem_h, add=True)
        return carry

    lax.fori_loop(0, CHUNKS, body, 0)

    def drain(j, carry):
        pltpu.make_async_copy(ones_v, acc_sh.at[idx_v.at[0]], sem_h).wait()
        return carry

    lax.fori_loop(0, CHUNKS, drain, 0)
    plsc.subcore_barrier()

    pltpu.sync_copy(acc_sh.at[pl.ds(s * RPT, RPT)], stage_v)
    pltpu.sync_copy(stage_v, deg_out.at[c, pl.ds(s * RPT, RPT)])


def _sc_bcast(v, l):
    return lax.gather(
        v, jnp.full((16, 1), l, jnp.int32),
        dimension_numbers=lax.GatherDimensionNumbers(
            offset_dims=(), collapsed_slice_dims=(0,), start_index_map=(0,)),
        slice_sizes=(1,), mode=lax.GatherScatterMode.PROMISE_IN_BOUNDS)


def _sc_rsqrt(d):
    ti = plsc.bitcast(d, jnp.int32)
    ti = jnp.full((16,), 0x5F3759DF, jnp.int32) - (ti >> 1)
    y = plsc.bitcast(ti, jnp.float32)
    for _ in range(3):
        y = y * (1.5 - 0.5 * d * y * y)
    return y


def _pipeline_prop(z_src, si_v, di_v, gbufs, acc_sh, semg, sems):
    for b in range(GA):
        pltpu.async_copy(z_src.at[si_v.at[b]], gbufs.at[b], semg.at[b])

    def group(g, carry):
        for b in range(NBUF):
            j = g * NBUF + b
            pltpu.make_async_copy(
                z_src.at[si_v.at[j]], gbufs.at[b], semg.at[b]).wait()
            pltpu.async_copy(
                gbufs.at[b], acc_sh.at[di_v.at[j]], sems.at[b], add=True)
            bc = (b + GA) % NBUF

            def refill():
                pltpu.make_async_copy(
                    gbufs.at[bc], acc_sh.at[di_v.at[j]], sems.at[bc]).wait()
                pltpu.async_copy(
                    z_src.at[si_v.at[j + GA]], gbufs.at[bc], semg.at[bc])

            if b < GA:
                @pl.when(g > 0)
                def _():
                    refill()

                @pl.when(g == 0)
                def _():
                    pltpu.async_copy(z_src.at[si_v.at[j + GA]],
                                     gbufs.at[bc], semg.at[bc])
            else:
                @pl.when(g < CHUNKS // NBUF - 1)
                def _():
                    refill()
        return carry

    lax.fori_loop(0, CHUNKS // NBUF, group, 0)
    for b in range(NBUF):
        pltpu.make_async_copy(
            gbufs.at[b], acc_sh.at[di_v.at[CHUNKS - 1]], sems.at[b]).wait()


def _writeback(out_hbm, c, s, gbufs, acc_sh, semg, sems):
    nwb = RPT // 128
    for i in range(nwb):
        base = s * RPT + i * 128
        pltpu.async_copy(acc_sh.at[pl.ds(base, 128)], gbufs.at[i], semg.at[i])
    for i in range(nwb):
        base = s * RPT + i * 128
        pltpu.make_async_copy(acc_sh.at[pl.ds(base, 128)], gbufs.at[i],
                              semg.at[i]).wait()
        pltpu.async_copy(gbufs.at[i], out_hbm.at[c, pl.ds(base, 128)],
                         sems.at[i])
    for i in range(nwb):
        base = s * RPT + i * 128
        pltpu.make_async_copy(gbufs.at[i], out_hbm.at[c, pl.ds(base, 128)],
                              sems.at[i]).wait()


@functools.partial(
    pl.kernel,
    mesh=_MESH,
    out_type=jax.ShapeDtypeStruct((2, N_PAD, H), jnp.float32),
    compiler_params=_SC_PARAMS_NL,
    scratch_types=[
        pltpu.VMEM((CHUNKS, 128), jnp.int32),
        pltpu.VMEM((CHUNKS, 128), jnp.int32),
        pltpu.VMEM((NBUF, 128, H), jnp.float32),
        pltpu.VMEM((128, H), jnp.float32),
        pltpu.VMEM((2, RPT), jnp.float32),
        pltpu.VMEM_SHARED((N_PAD, H), jnp.float32),
        pltpu.VMEM_SHARED((N_PAD, H), jnp.float32),
        pltpu.SemaphoreType.DMA((NBUF,)),
        pltpu.SemaphoreType.DMA((NBUF,)),
    ],
)
def _sc_prop1(y1_hbm, deg_hbm, src_hbm, dst_hbm, out_hbm, si_v, di_v, gbufs,
              zbuf, dbuf, acc_sh, z_sh, semg, sems):
    c = lax.axis_index("c")
    s = lax.axis_index("s")
    wid = s * 2 + c

    ld_si = pltpu.async_copy(src_hbm.at[wid], si_v, semg.at[0])
    ld_di = pltpu.async_copy(dst_hbm.at[wid], di_v, semg.at[1])
    ld_d0 = pltpu.async_copy(deg_hbm.at[0, pl.ds(s * RPT, RPT)], dbuf.at[0],
                             semg.at[2])
    ld_d1 = pltpu.async_copy(deg_hbm.at[1, pl.ds(s * RPT, RPT)], dbuf.at[1],
                             semg.at[3])
    for i in range(RPT // 128):
        pltpu.async_copy(y1_hbm.at[pl.ds(s * RPT + i * 128, 128)],
                         gbufs.at[i], sems.at[i])

    def zb(i, carry):
        zbuf[i, :] = jnp.zeros((16,), jnp.float32)
        return carry

    lax.fori_loop(0, 128, zb, 0)

    def zacc(i, carry):
        pltpu.sync_copy(zbuf, acc_sh.at[pl.ds(s * RPT + i * 128, 128)])
        return carry

    lax.fori_loop(0, RPT // 128, zacc, 0)
    ld_d0.wait()
    ld_d1.wait()

    for i in range(RPT // 128):
        pltpu.make_async_copy(y1_hbm.at[pl.ds(s * RPT + i * 128, 128)],
                              gbufs.at[i], sems.at[i]).wait()

        def scale_grp(r, carry):
            base16 = i * 128 + r * 16
            d16 = dbuf[0, pl.ds(base16, 16)] + dbuf[1, pl.ds(base16, 16)] + 1.0
            y = _sc_rsqrt(d16)
            for l in range(16):
                bc = _sc_bcast(y, l)
                gbufs[i, r * 16 + l, :] = gbufs[i, r * 16 + l, :] * bc
            return carry

        lax.fori_loop(0, 8, scale_grp, 0)
        pltpu.sync_copy(gbufs.at[i], z_sh.at[pl.ds(s * RPT + i * 128, 128)])

    ld_si.wait()
    ld_di.wait()
    plsc.subcore_barrier()
    _pipeline_prop(z_sh, si_v, di_v, gbufs, acc_sh, semg, sems)
    plsc.subcore_barrier()
    _writeback(out_hbm, c, s, gbufs, acc_sh, semg, sems)


@functools.partial(
    pl.kernel,
    mesh=_MESH,
    out_type=jax.ShapeDtypeStruct((2, N_PAD, CP), jnp.float32),
    compiler_params=_SC_PARAMS,
    scratch_types=[
        pltpu.VMEM((CHUNKS, 128), jnp.int32),
        pltpu.VMEM((CHUNKS, 128), jnp.int32),
        pltpu.VMEM((NBUF, 128, CP), jnp.float32),
        pltpu.VMEM((128, CP), jnp.float32),
        pltpu.VMEM_SHARED((N_PAD, CP), jnp.float32),
        pltpu.SemaphoreType.DMA((NBUF,)),
        pltpu.SemaphoreType.DMA((NBUF,)),
    ],
)
def _sc_prop2(z_hbm, src_hbm, dst_hbm, out_hbm, si_v, di_v, gbufs, zbuf,
              acc_sh, semg, sems):
    c = lax.axis_index("c")
    s = lax.axis_index("s")
    wid = s * 2 + c

    ld_si = pltpu.async_copy(src_hbm.at[wid], si_v, semg.at[0])
    ld_di = pltpu.async_copy(dst_hbm.at[wid], di_v, semg.at[1])

    def zb(i, carry):
        for k in range(CP // 16):
            zbuf[i, pl.ds(k * 16, 16)] = jnp.zeros((16,), jnp.float32)
        return carry

    lax.fori_loop(0, 128, zb, 0)

    def zacc(i, carry):
        pltpu.sync_copy(zbuf, acc_sh.at[pl.ds(s * RPT + i * 128, 128)])
        return carry

    lax.fori_loop(0, RPT // 128, zacc, 0)
    ld_si.wait()
    ld_di.wait()
    plsc.subcore_barrier()
    _pipeline_prop(z_hbm, si_v, di_v, gbufs, acc_sh, semg, sems)
    plsc.subcore_barrier()
    _writeback(out_hbm, c, s, gbufs, acc_sh, semg, sems)


def _dinv_from(degT_ref):
    d = degT_ref[:, 0:1] + degT_ref[:, 1:2] + 1.0
    rows = lax.broadcasted_iota(jnp.int32, (N_PAD, 1), 0)
    return jnp.where(rows < N, lax.rsqrt(d), 0.0)


def _tc_matmul1_body(x_ref, w1_ref, y1_ref):
    h = jnp.dot(x_ref[...], w1_ref[...], preferred_element_type=jnp.float32)
    y1_ref[pl.ds(0, N), :] = h
    y1_ref[pl.ds(N, N_PAD - N), :] = jnp.zeros((N_PAD - N, H), jnp.float32)


def _tc_layer2_body(s1_ref, y1_ref, degT_ref, b1_ref, w2_ref, z2_ref):
    dinv = _dinv_from(degT_ref)
    z1 = dinv * y1_ref[...]
    t = dinv * (s1_ref[0] + s1_ref[1] + z1) + b1_ref[...]
    out1 = jnp.maximum(t, 0.0)
    z2_ref[...] = dinv * jnp.dot(out1, w2_ref[...],
                                 preferred_element_type=jnp.float32)


def _tc_softmax_body(s2_ref, z2_ref, degT_ref, b2_ref, out_ref):
    dinv = _dinv_from(degT_ref)
    u = dinv * (s2_ref[0] + s2_ref[1] + z2_ref[...]) + b2_ref[...]
    cols = lax.broadcasted_iota(jnp.int32, (1, CP), 1)
    cmask = cols < C
    u = jnp.where(cmask, u, -jnp.inf)
    m = jnp.max(u, axis=1, keepdims=True)
    e = jnp.where(cmask, jnp.exp(u - m), 0.0)
    p = e / jnp.sum(e, axis=1, keepdims=True)
    out_ref[...] = p[:N, :C]


def kernel(x, edge_index, W1, b1, W2, b2):
    src = edge_index[0]
    dst = edge_index[1]
    npad_e = E_PAD - E
    fill = N + (jnp.arange(npad_e, dtype=jnp.int32) % N_DUMMY_ROWS)
    src3 = jnp.concatenate([src, fill]).reshape(NW, CHUNKS, 128)
    dst3 = jnp.concatenate([dst, fill]).reshape(NW, CHUNKS, 128)
    w2p = jnp.zeros((H, CP), jnp.float32).at[:, :C].set(W2)
    b1r = b1.reshape(1, H)
    b2r = jnp.zeros((1, CP), jnp.float32).at[0, :C].set(b2)

    deg = _sc_degree(dst3)
    y1 = pl.pallas_call(
        _tc_matmul1_body,
        out_shape=jax.ShapeDtypeStruct((N_PAD, H), jnp.float32),
    )(x, W1)

    s1 = _sc_prop1(y1, deg, src3, dst3)
    degT = deg.T

    z2 = pl.pallas_call(
        _tc_layer2_body,
        out_shape=jax.ShapeDtypeStruct((N_PAD, CP), jnp.float32),
    )(s1, y1, degT, b1r, w2p)

    s2 = _sc_prop2(z2, src3, dst3)

    return pl.pallas_call(
        _tc_softmax_body,
        out_shape=jax.ShapeDtypeStruct((N, C), jnp.float32),
    )(s2, z2, degT, b2r)


# speedup vs baseline: 61.0525x; 1.0030x over previous
import functools

import jax
import jax.numpy as jnp
from jax import lax
from jax.experimental import pallas as pl
from jax.experimental.pallas import tpu as pltpu
from jax.experimental.pallas import tpu_sc as plsc

N = 10000
E = 320000
D_IN = 128
H = 16
C = 40
CP = 48

NW = 32
N_PAD = 10240
RPT = N_PAD // 16
E_PAD = 327680
CHUNKS = E_PAD // (NW * 128)
N_DUMMY_ROWS = 128

_MESH = plsc.VectorSubcoreMesh(core_axis_name="c", subcore_axis_name="s")
_SC_PARAMS = pltpu.CompilerParams(use_tc_tiling_on_sc=False)
_SC_PARAMS_NL = pltpu.CompilerParams(use_tc_tiling_on_sc=False,
                                     needs_layout_passes=False)

NBUF = 10
GA = 5


@functools.partial(
    pl.kernel,
    mesh=_MESH,
    out_type=jax.ShapeDtypeStruct((2, N_PAD), jnp.float32),
    compiler_params=_SC_PARAMS,
    scratch_types=[
        pltpu.VMEM((CHUNKS, 128), jnp.int32),
        pltpu.VMEM((128,), jnp.float32),
        pltpu.VMEM((RPT,), jnp.float32),
        pltpu.VMEM_SHARED((N_PAD,), jnp.float32),
        pltpu.SemaphoreType.DMA,
    ],
)
def _sc_degree(dst_hbm, deg_out, idx_v, ones_v, stage_v, acc_sh, sem_h):
    c = lax.axis_index("c")
    s = lax.axis_index("s")
    wid = s * 2 + c

    def zstage(i, carry):
        stage_v[pl.ds(i * 16, 16)] = jnp.zeros((16,), jnp.float32)
        return carry

    lax.fori_loop(0, RPT // 16, zstage, 0)
    pltpu.sync_copy(stage_v, acc_sh.at[pl.ds(s * RPT, RPT)])
    for k in range(8):
        ones_v[pl.ds(k * 16, 16)] = jnp.ones((16,), jnp.float32)
    plsc.subcore_barrier()

    pltpu.sync_copy(dst_hbm.at[wid], idx_v)

    def body(j, carry):
        pltpu.async_copy(ones_v, acc_sh.at[idx_v.at[j]], sem_h, add=True)
        return carry

    lax.fori_loop(0, CHUNKS, body, 0)

    def drain(j, carry):
        pltpu.make_async_copy(ones_v, acc_sh.at[idx_v.at[0]], sem_h).wait()
        return carry

    lax.fori_loop(0, CHUNKS, drain, 0)
    plsc.subcore_barrier()

    pltpu.sync_copy(acc_sh.at[pl.ds(s * RPT, RPT)], stage_v)
    pltpu.sync_copy(stage_v, deg_out.at[c, pl.ds(s * RPT, RPT)])


def _sc_bcast(v, l):
    return lax.gather(
        v, jnp.full((16, 1), l, jnp.int32),
        dimension_numbers=lax.GatherDimensionNumbers(
            offset_dims=(), collapsed_slice_dims=(0,), start_index_map=(0,)),
        slice_sizes=(1,), mode=lax.GatherScatterMode.PROMISE_IN_BOUNDS)


def _sc_rsqrt(d):
    ti = plsc.bitcast(d, jnp.int32)
    ti = jnp.full((16,), 0x5F3759DF, jnp.int32) - (ti >> 1)
    y = plsc.bitcast(ti, jnp.float32)
    for _ in range(3):
        y = y * (1.5 - 0.5 * d * y * y)
    return y


def _pipeline_prop(z_src, si_v, di_v, gbufs, acc_sh, semg, sems):
    for b in range(GA):
        pltpu.async_copy(z_src.at[si_v.at[b]], gbufs.at[b], semg.at[b])

    def group(g, carry):
        for b in range(NBUF):
            j = g * NBUF + b
            pltpu.make_async_copy(
                z_src.at[si_v.at[j]], gbufs.at[b], semg.at[b]).wait()
            pltpu.async_copy(
                gbufs.at[b], acc_sh.at[di_v.at[j]], sems.at[b], add=True)
            bc = (b + GA) % NBUF

            def refill():
                pltpu.make_async_copy(
                    gbufs.at[bc], acc_sh.at[di_v.at[j]], sems.at[bc]).wait()
                pltpu.async_copy(
                    z_src.at[si_v.at[j + GA]], gbufs.at[bc], semg.at[bc])

            if b < GA:
                @pl.when(g > 0)
                def _():
                    refill()

                @pl.when(g == 0)
                def _():
                    pltpu.async_copy(z_src.at[si_v.at[j + GA]],
                                     gbufs.at[bc], semg.at[bc])
            else:
                @pl.when(g < CHUNKS // NBUF - 1)
                def _():
                    refill()
        return carry

    lax.fori_loop(0, CHUNKS // NBUF, group, 0)
    for b in range(NBUF):
        pltpu.make_async_copy(
            gbufs.at[b], acc_sh.at[di_v.at[CHUNKS - 1]], sems.at[b]).wait()


def _writeback(out_hbm, c, s, gbufs, acc_sh, semg, sems):
    nwb = RPT // 128
    for i in range(nwb):
        base = s * RPT + i * 128
        pltpu.async_copy(acc_sh.at[pl.ds(base, 128)], gbufs.at[i], semg.at[i])
    for i in range(nwb):
        base = s * RPT + i * 128
        pltpu.make_async_copy(acc_sh.at[pl.ds(base, 128)], gbufs.at[i],
                              semg.at[i]).wait()
        pltpu.async_copy(gbufs.at[i], out_hbm.at[c, pl.ds(base, 128)],
                         sems.at[i])
    for i in range(nwb):
        base = s * RPT + i * 128
        pltpu.make_async_copy(gbufs.at[i], out_hbm.at[c, pl.ds(base, 128)],
                              sems.at[i]).wait()


@functools.partial(
    pl.kernel,
    mesh=_MESH,
    out_type=jax.ShapeDtypeStruct((2, N_PAD, H), jnp.float32),
    compiler_params=_SC_PARAMS_NL,
    scratch_types=[
        pltpu.VMEM((CHUNKS, 128), jnp.int32),
        pltpu.VMEM((CHUNKS, 128), jnp.int32),
        pltpu.VMEM((NBUF, 128, H), jnp.float32),
        pltpu.VMEM((128, H), jnp.float32),
        pltpu.VMEM((2, RPT), jnp.float32),
        pltpu.VMEM_SHARED((N_PAD, H), jnp.float32),
        pltpu.VMEM_SHARED((N_PAD, H), jnp.float32),
        pltpu.SemaphoreType.DMA((NBUF,)),
        pltpu.SemaphoreType.DMA((NBUF,)),
    ],
)
def _sc_prop1(y1_hbm, deg_hbm, src_hbm, dst_hbm, out_hbm, si_v, di_v, gbufs,
              zbuf, dbuf, acc_sh, z_sh, semg, sems):
    c = lax.axis_index("c")
    s = lax.axis_index("s")
    wid = s * 2 + c

    ld_si = pltpu.async_copy(src_hbm.at[wid], si_v, semg.at[0])
    ld_di = pltpu.async_copy(dst_hbm.at[wid], di_v, semg.at[1])
    ld_d0 = pltpu.async_copy(deg_hbm.at[0, pl.ds(s * RPT, RPT)], dbuf.at[0],
                             semg.at[2])
    ld_d1 = pltpu.async_copy(deg_hbm.at[1, pl.ds(s * RPT, RPT)], dbuf.at[1],
                             semg.at[3])
    for i in range(RPT // 128):
        pltpu.async_copy(y1_hbm.at[pl.ds(s * RPT + i * 128, 128)],
                         gbufs.at[i], sems.at[i])

    def zb(i, carry):
        zbuf[i, :] = jnp.zeros((16,), jnp.float32)
        return carry

    lax.fori_loop(0, 128, zb, 0)

    for i in range(RPT // 128):
        pltpu.async_copy(zbuf, acc_sh.at[pl.ds(s * RPT + i * 128, 128)],
                         semg.at[4])
    ld_d0.wait()
    ld_d1.wait()

    for i in range(RPT // 128):
        pltpu.make_async_copy(y1_hbm.at[pl.ds(s * RPT + i * 128, 128)],
                              gbufs.at[i], sems.at[i]).wait()

        def scale_grp(r, carry):
            base16 = i * 128 + r * 16
            d16 = dbuf[0, pl.ds(base16, 16)] + dbuf[1, pl.ds(base16, 16)] + 1.0
            y = _sc_rsqrt(d16)
            for l in range(16):
                bc = _sc_bcast(y, l)
                gbufs[i, r * 16 + l, :] = gbufs[i, r * 16 + l, :] * bc
            return carry

        lax.fori_loop(0, 8, scale_grp, 0)
        pltpu.async_copy(gbufs.at[i], z_sh.at[pl.ds(s * RPT + i * 128, 128)],
                         sems.at[i])

    for i in range(RPT // 128):
        pltpu.make_async_copy(zbuf, acc_sh.at[pl.ds(s * RPT + i * 128, 128)],
                              semg.at[4]).wait()
        pltpu.make_async_copy(gbufs.at[i],
                              z_sh.at[pl.ds(s * RPT + i * 128, 128)],
                              sems.at[i]).wait()
    ld_si.wait()
    ld_di.wait()
    plsc.subcore_barrier()
    _pipeline_prop(z_sh, si_v, di_v, gbufs, acc_sh, semg, sems)
    plsc.subcore_barrier()
    _writeback(out_hbm, c, s, gbufs, acc_sh, semg, sems)


@functools.partial(
    pl.kernel,
    mesh=_MESH,
    out_type=jax.ShapeDtypeStruct((2, N_PAD, CP), jnp.float32),
    compiler_params=_SC_PARAMS,
    scratch_types=[
        pltpu.VMEM((CHUNKS, 128), jnp.int32),
        pltpu.VMEM((CHUNKS, 128), jnp.int32),
        pltpu.VMEM((NBUF, 128, CP), jnp.float32),
        pltpu.VMEM((128, CP), jnp.float32),
        pltpu.VMEM_SHARED((N_PAD, CP), jnp.float32),
        pltpu.SemaphoreType.DMA((NBUF,)),
        pltpu.SemaphoreType.DMA((NBUF,)),
    ],
)
def _sc_prop2(z_hbm, src_hbm, dst_hbm, out_hbm, si_v, di_v, gbufs, zbuf,
              acc_sh, semg, sems):
    c = lax.axis_index("c")
    s = lax.axis_index("s")
    wid = s * 2 + c

    ld_si = pltpu.async_copy(src_hbm.at[wid], si_v, semg.at[0])
    ld_di = pltpu.async_copy(dst_hbm.at[wid], di_v, semg.at[1])

    def zb(i, carry):
        for k in range(CP // 16):
            zbuf[i, pl.ds(k * 16, 16)] = jnp.zeros((16,), jnp.float32)
        return carry

    lax.fori_loop(0, 128, zb, 0)

    for i in range(RPT // 128):
        pltpu.async_copy(zbuf, acc_sh.at[pl.ds(s * RPT + i * 128, 128)],
                         semg.at[2])
    for i in range(RPT // 128):
        pltpu.make_async_copy(zbuf, acc_sh.at[pl.ds(s * RPT + i * 128, 128)],
                              semg.at[2]).wait()
    ld_si.wait()
    ld_di.wait()
    plsc.subcore_barrier()
    _pipeline_prop(z_hbm, si_v, di_v, gbufs, acc_sh, semg, sems)
    plsc.subcore_barrier()
    _writeback(out_hbm, c, s, gbufs, acc_sh, semg, sems)


def _dinv_from(degT_ref):
    d = degT_ref[:, 0:1] + degT_ref[:, 1:2] + 1.0
    rows = lax.broadcasted_iota(jnp.int32, (N_PAD, 1), 0)
    return jnp.where(rows < N, lax.rsqrt(d), 0.0)


def _tc_matmul1_body(x_ref, w1_ref, y1_ref):
    h = jnp.dot(x_ref[...], w1_ref[...], preferred_element_type=jnp.float32)
    y1_ref[pl.ds(0, N), :] = h
    y1_ref[pl.ds(N, N_PAD - N), :] = jnp.zeros((N_PAD - N, H), jnp.float32)


def _tc_layer2_body(s1_ref, y1_ref, degT_ref, b1_ref, w2_ref, z2_ref):
    dinv = _dinv_from(degT_ref)
    z1 = dinv * y1_ref[...]
    t = dinv * (s1_ref[0] + s1_ref[1] + z1) + b1_ref[...]
    out1 = jnp.maximum(t, 0.0)
    z2_ref[...] = dinv * jnp.dot(out1, w2_ref[...],
                                 preferred_element_type=jnp.float32)


def _tc_softmax_body(s2_ref, z2_ref, degT_ref, b2_ref, out_ref):
    dinv = _dinv_from(degT_ref)
    u = dinv * (s2_ref[0] + s2_ref[1] + z2_ref[...]) + b2_ref[...]
    cols = lax.broadcasted_iota(jnp.int32, (1, CP), 1)
    cmask = cols < C
    u = jnp.where(cmask, u, -jnp.inf)
    m = jnp.max(u, axis=1, keepdims=True)
    e = jnp.where(cmask, jnp.exp(u - m), 0.0)
    p = e / jnp.sum(e, axis=1, keepdims=True)
    out_ref[...] = p[:N, :C]


def kernel(x, edge_index, W1, b1, W2, b2):
    src = edge_index[0]
    dst = edge_index[1]
    npad_e = E_PAD - E
    fill = N + (jnp.arange(npad_e, dtype=jnp.int32) % N_DUMMY_ROWS)
    src3 = jnp.concatenate([src, fill]).reshape(NW, CHUNKS, 128)
    dst3 = jnp.concatenate([dst, fill]).reshape(NW, CHUNKS, 128)
    w2p = jnp.zeros((H, CP), jnp.float32).at[:, :C].set(W2)
    b1r = b1.reshape(1, H)
    b2r = jnp.zeros((1, CP), jnp.float32).at[0, :C].set(b2)

    deg = _sc_degree(dst3)
    y1 = pl.pallas_call(
        _tc_matmul1_body,
        out_shape=jax.ShapeDtypeStruct((N_PAD, H), jnp.float32),
    )(x, W1)

    s1 = _sc_prop1(y1, deg, src3, dst3)
    degT = deg.T

    z2 = pl.pallas_call(
        _tc_layer2_body,
        out_shape=jax.ShapeDtypeStruct((N_PAD, CP), jnp.float32),
    )(s1, y1, degT, b1r, w2p)

    s2 = _sc_prop2(z2, src3, dst3)

    return pl.pallas_call(
        _tc_softmax_body,
        out_shape=jax.ShapeDtypeStruct((N, C), jnp.float32),
    )(s2, z2, degT, b2r)
